# baseline (speedup 1.0000x reference)
import functools

import jax
import jax.numpy as jnp
from jax import lax
from jax.experimental import pallas as pl
from jax.experimental.pallas import tpu as pltpu
from jax.experimental.pallas import tpu_sc as plsc

N = 10000
E = 320000
D = 128

NC = 2
NS = 16
NW = NC * NS
EB = 125
NBLK = E // EB
BPT = NBLK // NW
CHUNK = 40
CE = CHUNK * EB
NRING = 2
ROWS_PER_TILE = 624
REM_ROWS = N - NS * ROWS_PER_TILE
ZC = 120

_sc_mesh = plsc.VectorSubcoreMesh(
    core_axis_name="c", subcore_axis_name="s", num_cores=NC, num_subcores=NS)


@functools.partial(
    pl.kernel,
    out_type=jax.ShapeDtypeStruct((2 * N, D), jnp.float32),
    mesh=_sc_mesh,
    scratch_types=[
        pltpu.VMEM_SHARED((N, D), jnp.float32),
        pltpu.VMEM((CHUNK, EB), jnp.int32),
        pltpu.VMEM((CHUNK, EB), jnp.int32),
        pltpu.VMEM((NRING, EB, D), jnp.float32),
        pltpu.SemaphoreType.DMA,
        pltpu.SemaphoreType.DMA,
        pltpu.SemaphoreType.DMA,
        pltpu.SemaphoreType.DMA,
    ],
)
def _segsum_sc(x_hbm, ei_hbm, out_hbm, acc, sidx, didx, rows, g0, g1,
               s0, s1):
    cid = lax.axis_index("c")
    sid = lax.axis_index("s")
    wid = sid * NC + cid
    gsems = (g0, g1)
    ssems = (s0, s1)
    blk0 = wid * BPT
    r0 = sid * ROWS_PER_TILE

    @pl.when(cid == 0)
    def _init_x():
        pltpu.sync_copy(x_hbm.at[pl.ds(r0, ROWS_PER_TILE)],
                        acc.at[pl.ds(r0, ROWS_PER_TILE)])

        @pl.when(sid == NS - 1)
        def _init_rem():
            pltpu.sync_copy(x_hbm.at[pl.ds(NS * ROWS_PER_TILE, REM_ROWS)],
                            acc.at[pl.ds(NS * ROWS_PER_TILE, REM_ROWS)])

    @pl.when(cid == 1)
    def _init_zero():
        def _zrow(k, carry):
            i = k // (D // 16)
            j = k - i * (D // 16)
            rows[0, i, pl.ds(j * 16, 16)] = jnp.zeros((16,), jnp.float32)
            return carry
        lax.fori_loop(0, ZC * (D // 16), _zrow, 0)
        z0 = rows.at[0]
        for k in range(ROWS_PER_TILE // ZC):
            pltpu.sync_copy(z0.at[pl.ds(0, ZC)],
                            acc.at[pl.ds(r0 + k * ZC, ZC)])
        pltpu.sync_copy(z0.at[pl.ds(0, ROWS_PER_TILE - 5 * ZC)],
                        acc.at[pl.ds(r0 + 5 * ZC, ROWS_PER_TILE - 5 * ZC)])

        @pl.when(sid == NS - 1)
        def _zero_rem():
            pltpu.sync_copy(z0.at[pl.ds(0, REM_ROWS)],
                            acc.at[pl.ds(NS * ROWS_PER_TILE, REM_ROWS)])

    plsc.subcore_barrier()

    for half in range(BPT // CHUNK):
        base = blk0 + half * CHUNK
        pltpu.sync_copy(ei_hbm.at[0, pl.ds(base, CHUNK)], sidx)
        pltpu.sync_copy(ei_hbm.at[1, pl.ds(base, CHUNK)], didx)
        for b in range(NRING):
            pltpu.async_copy(x_hbm.at[sidx.at[b]], rows.at[b], gsems[b])

        def _pair(p, carry):
            for b in range(NRING):
                j = p * NRING + b
                pltpu.make_async_copy(x_hbm.at[sidx.at[j]], rows.at[b],
                                      gsems[b]).wait()
                pltpu.async_copy(rows.at[b], acc.at[didx.at[j]], ssems[b],
                                 add=True)
            for b in range(NRING):
                j = p * NRING + b
                pltpu.make_async_copy(rows.at[b], acc.at[didx.at[j]],
                                      ssems[b]).wait()
                pltpu.async_copy(x_hbm.at[sidx.at[j + NRING]], rows.at[b],
                                 gsems[b])
            return carry
        lax.fori_loop(0, CHUNK // NRING - 1, _pair, 0)
        for b in range(NRING):
            j = CHUNK - NRING + b
            pltpu.make_async_copy(x_hbm.at[sidx.at[j]], rows.at[b],
                                  gsems[b]).wait()
            pltpu.async_copy(rows.at[b], acc.at[didx.at[j]], ssems[b],
                             add=True)
        for b in range(NRING):
            j = CHUNK - NRING + b
            pltpu.make_async_copy(rows.at[b], acc.at[didx.at[j]],
                                  ssems[b]).wait()

    plsc.subcore_barrier()

    pltpu.sync_copy(acc.at[pl.ds(r0, ROWS_PER_TILE)],
                    out_hbm.at[pl.ds(cid * N + r0, ROWS_PER_TILE)])

    @pl.when(sid == NS - 1)
    def _pub_rem():
        pltpu.sync_copy(acc.at[pl.ds(NS * ROWS_PER_TILE, REM_ROWS)],
                        out_hbm.at[pl.ds(cid * N + NS * ROWS_PER_TILE,
                                         REM_ROWS)])


def _mlp_body(with_post_bn, p_ref, w1_ref, b1_ref, g1_ref, be1_ref,
              w2_ref, b2_ref, g3_ref, be3_ref, o_ref):
    u = p_ref[0:N, :] + p_ref[N:2 * N, :]
    t = jnp.dot(u, w1_ref[...], preferred_element_type=jnp.float32)
    t = t + b1_ref[...]
    mean = jnp.mean(t, axis=0, keepdims=True)
    var = jnp.mean((t - mean) ** 2, axis=0, keepdims=True)
    t = (t - mean) / jnp.sqrt(var + 1e-5) * g1_ref[...] + be1_ref[...]
    t = jnp.maximum(t, 0.0)
    t = jnp.dot(t, w2_ref[...], preferred_element_type=jnp.float32)
    t = t + b2_ref[...]
    if with_post_bn:
        mean = jnp.mean(t, axis=0, keepdims=True)
        var = jnp.mean((t - mean) ** 2, axis=0, keepdims=True)
        t = (t - mean) / jnp.sqrt(var + 1e-5) * g3_ref[...] + be3_ref[...]
        t = jnp.maximum(t, 0.0)
    o_ref[...] = t


def _mlp_tc(parts, w1, b1, g1, be1, w2, b2, g3, be3, with_post_bn):
    r2 = lambda v: v.reshape(1, D)
    return pl.pallas_call(
        functools.partial(_mlp_body, with_post_bn),
        out_shape=jax.ShapeDtypeStruct((N, D), jnp.float32),
    )(parts, w1, r2(b1), r2(g1), r2(be1), w2, r2(b2), r2(g3), r2(be3))


def kernel(x, edge_index, w0_1, b0_1, g0_1, be0_1, w0_2, b0_2, g0_3, be0_3,
           w1_1, b1_1, g1_1, be1_1, w1_2, b1_2):
    ei = edge_index.reshape(2, NBLK, EB)
    parts0 = _segsum_sc(x, ei)
    h = _mlp_tc(parts0, w0_1, b0_1, g0_1, be0_1, w0_2, b0_2, g0_3, be0_3,
                with_post_bn=True)
    parts1 = _segsum_sc(h, ei)
    out = _mlp_tc(parts1, w1_1, b1_1, g1_1, be1_1, w1_2, b1_2, g1_1, be1_1,
                  with_post_bn=False)
    return out


# speedup vs baseline: 1.2675x; 1.2675x over previous
import functools

import jax
import jax.numpy as jnp
from jax import lax
from jax.experimental import pallas as pl
from jax.experimental.pallas import tpu as pltpu
from jax.experimental.pallas import tpu_sc as plsc

N = 10000
E = 320000
D = 128

NC = 2
NS = 16
NW = NC * NS
EB = 125
NBLK = E // EB
BPT = NBLK // NW
CHUNK = 40
CE = CHUNK * EB
NRING = 2
ROWS_PER_TILE = 624
REM_ROWS = N - NS * ROWS_PER_TILE
ZC = 120

_sc_mesh = plsc.VectorSubcoreMesh(
    core_axis_name="c", subcore_axis_name="s", num_cores=NC, num_subcores=NS)


@functools.partial(
    pl.kernel,
    out_type=jax.ShapeDtypeStruct((2 * N, D), jnp.float32),
    mesh=_sc_mesh,
    scratch_types=[
        pltpu.VMEM_SHARED((N, D), jnp.float32),
        pltpu.VMEM((CHUNK, EB), jnp.int32),
        pltpu.VMEM((CHUNK, EB), jnp.int32),
        pltpu.VMEM((NRING, EB, D), jnp.float32),
        pltpu.SemaphoreType.DMA,
        pltpu.SemaphoreType.DMA,
    ],
)
def _segsum_sc(x_hbm, ei_hbm, out_hbm, acc, sidx, didx, rows, g0, g1):
    cid = lax.axis_index("c")
    sid = lax.axis_index("s")
    wid = sid * NC + cid
    gsems = (g0, g1)
    blk0 = wid * BPT
    r0 = sid * ROWS_PER_TILE

    @pl.when(cid == 0)
    def _init_x():
        pltpu.sync_copy(x_hbm.at[pl.ds(r0, ROWS_PER_TILE)],
                        acc.at[pl.ds(r0, ROWS_PER_TILE)])

        @pl.when(sid == NS - 1)
        def _init_rem():
            pltpu.sync_copy(x_hbm.at[pl.ds(NS * ROWS_PER_TILE, REM_ROWS)],
                            acc.at[pl.ds(NS * ROWS_PER_TILE, REM_ROWS)])

    @pl.when(cid == 1)
    def _init_zero():
        def _zrow(k, carry):
            i = k // (D // 16)
            j = k - i * (D // 16)
            rows[0, i, pl.ds(j * 16, 16)] = jnp.zeros((16,), jnp.float32)
            return carry
        lax.fori_loop(0, ZC * (D // 16), _zrow, 0)
        z0 = rows.at[0]
        for k in range(ROWS_PER_TILE // ZC):
            pltpu.sync_copy(z0.at[pl.ds(0, ZC)],
                            acc.at[pl.ds(r0 + k * ZC, ZC)])
        pltpu.sync_copy(z0.at[pl.ds(0, ROWS_PER_TILE - 5 * ZC)],
                        acc.at[pl.ds(r0 + 5 * ZC, ROWS_PER_TILE - 5 * ZC)])

        @pl.when(sid == NS - 1)
        def _zero_rem():
            pltpu.sync_copy(z0.at[pl.ds(0, REM_ROWS)],
                            acc.at[pl.ds(NS * ROWS_PER_TILE, REM_ROWS)])

    plsc.subcore_barrier()

    for half in range(BPT // CHUNK):
        base = blk0 + half * CHUNK
        pltpu.sync_copy(ei_hbm.at[0, pl.ds(base, CHUNK)], sidx)
        pltpu.sync_copy(ei_hbm.at[1, pl.ds(base, CHUNK)], didx)
        for b in range(NRING):
            pltpu.async_copy(x_hbm.at[sidx.at[b]], rows.at[b], gsems[b])

        def _pair(p, carry):
            for b in range(NRING):
                j = p * NRING + b
                pltpu.make_async_copy(x_hbm.at[sidx.at[j]], rows.at[b],
                                      gsems[b]).wait()
                pltpu.sync_copy(rows.at[b], acc.at[didx.at[j]], add=True)
                pltpu.async_copy(x_hbm.at[sidx.at[j + NRING]], rows.at[b],
                                 gsems[b])
            return carry
        lax.fori_loop(0, CHUNK // NRING - 1, _pair, 0)
        for b in range(NRING):
            j = CHUNK - NRING + b
            pltpu.make_async_copy(x_hbm.at[sidx.at[j]], rows.at[b],
                                  gsems[b]).wait()
            pltpu.sync_copy(rows.at[b], acc.at[didx.at[j]], add=True)

    plsc.subcore_barrier()

    pltpu.sync_copy(acc.at[pl.ds(r0, ROWS_PER_TILE)],
                    out_hbm.at[pl.ds(cid * N + r0, ROWS_PER_TILE)])

    @pl.when(sid == NS - 1)
    def _pub_rem():
        pltpu.sync_copy(acc.at[pl.ds(NS * ROWS_PER_TILE, REM_ROWS)],
                        out_hbm.at[pl.ds(cid * N + NS * ROWS_PER_TILE,
                                         REM_ROWS)])


def _mlp_body(with_post_bn, p_ref, w1_ref, b1_ref, g1_ref, be1_ref,
              w2_ref, b2_ref, g3_ref, be3_ref, o_ref):
    u = p_ref[0:N, :] + p_ref[N:2 * N, :]
    t = jnp.dot(u, w1_ref[...], preferred_element_type=jnp.float32)
    t = t + b1_ref[...]
    mean = jnp.mean(t, axis=0, keepdims=True)
    var = jnp.mean((t - mean) ** 2, axis=0, keepdims=True)
    t = (t - mean) / jnp.sqrt(var + 1e-5) * g1_ref[...] + be1_ref[...]
    t = jnp.maximum(t, 0.0)
    t = jnp.dot(t, w2_ref[...], preferred_element_type=jnp.float32)
    t = t + b2_ref[...]
    if with_post_bn:
        mean = jnp.mean(t, axis=0, keepdims=True)
        var = jnp.mean((t - mean) ** 2, axis=0, keepdims=True)
        t = (t - mean) / jnp.sqrt(var + 1e-5) * g3_ref[...] + be3_ref[...]
        t = jnp.maximum(t, 0.0)
    o_ref[...] = t


def _mlp_tc(parts, w1, b1, g1, be1, w2, b2, g3, be3, with_post_bn):
    r2 = lambda v: v.reshape(1, D)
    return pl.pallas_call(
        functools.partial(_mlp_body, with_post_bn),
        out_shape=jax.ShapeDtypeStruct((N, D), jnp.float32),
    )(parts, w1, r2(b1), r2(g1), r2(be1), w2, r2(b2), r2(g3), r2(be3))


def kernel(x, edge_index, w0_1, b0_1, g0_1, be0_1, w0_2, b0_2, g0_3, be0_3,
           w1_1, b1_1, g1_1, be1_1, w1_2, b1_2):
    ei = edge_index.reshape(2, NBLK, EB)
    parts0 = _segsum_sc(x, ei)
    h = _mlp_tc(parts0, w0_1, b0_1, g0_1, be0_1, w0_2, b0_2, g0_3, be0_3,
                with_post_bn=True)
    parts1 = _segsum_sc(h, ei)
    out = _mlp_tc(parts1, w1_1, b1_1, g1_1, be1_1, w1_2, b1_2, g1_1, be1_1,
                  with_post_bn=False)
    return out


# speedup vs baseline: 1.2678x; 1.0003x over previous
import functools

import jax
import jax.numpy as jnp
from jax import lax
from jax.experimental import pallas as pl
from jax.experimental.pallas import tpu as pltpu
from jax.experimental.pallas import tpu_sc as plsc

N = 10000
E = 320000
D = 128

NC = 2
NS = 16
NW = NC * NS
EB = 125
NBLK = E // EB
BPT = NBLK // NW
CHUNK = 40
NRING = 2
ROWS_PER_TILE = 624
REM_ROWS = N - NS * ROWS_PER_TILE
ZC = 120

_sc_mesh = plsc.VectorSubcoreMesh(
    core_axis_name="c", subcore_axis_name="s", num_cores=NC, num_subcores=NS)


@functools.partial(
    pl.kernel,
    out_type=jax.ShapeDtypeStruct((2 * N, D), jnp.float32),
    mesh=_sc_mesh,
    scratch_types=[
        pltpu.VMEM_SHARED((N, D), jnp.float32),
        pltpu.VMEM((CHUNK, EB), jnp.int32),
        pltpu.VMEM((CHUNK, EB), jnp.int32),
        pltpu.VMEM((NRING, EB, D), jnp.float32),
        pltpu.SemaphoreType.DMA,
        pltpu.SemaphoreType.DMA,
    ],
)
def _segsum_sc(x_hbm, ei_hbm, out_hbm, acc, sidx, didx, rows, g0, g1):
    cid = lax.axis_index("c")
    sid = lax.axis_index("s")
    wid = sid * NC + cid
    gsems = (g0, g1)
    blk0 = wid * BPT
    r0 = sid * ROWS_PER_TILE

    @pl.when(cid == 0)
    def _init_x():
        pltpu.sync_copy(x_hbm.at[pl.ds(r0, ROWS_PER_TILE)],
                        acc.at[pl.ds(r0, ROWS_PER_TILE)])

        @pl.when(sid == NS - 1)
        def _init_rem():
            pltpu.sync_copy(x_hbm.at[pl.ds(NS * ROWS_PER_TILE, REM_ROWS)],
                            acc.at[pl.ds(NS * ROWS_PER_TILE, REM_ROWS)])

    @pl.when(cid == 1)
    def _init_zero():
        def _zrow(k, carry):
            i = k // (D // 16)
            j = k - i * (D // 16)
            rows[0, i, pl.ds(j * 16, 16)] = jnp.zeros((16,), jnp.float32)
            return carry
        lax.fori_loop(0, ZC * (D // 16), _zrow, 0)
        z0 = rows.at[0]
        for k in range(ROWS_PER_TILE // ZC):
            pltpu.sync_copy(z0.at[pl.ds(0, ZC)],
                            acc.at[pl.ds(r0 + k * ZC, ZC)])
        pltpu.sync_copy(z0.at[pl.ds(0, ROWS_PER_TILE - 5 * ZC)],
                        acc.at[pl.ds(r0 + 5 * ZC, ROWS_PER_TILE - 5 * ZC)])

        @pl.when(sid == NS - 1)
        def _zero_rem():
            pltpu.sync_copy(z0.at[pl.ds(0, REM_ROWS)],
                            acc.at[pl.ds(NS * ROWS_PER_TILE, REM_ROWS)])

    plsc.subcore_barrier()

    for half in range(BPT // CHUNK):
        base = blk0 + half * CHUNK
        pltpu.sync_copy(ei_hbm.at[0, pl.ds(base, CHUNK)], sidx)
        pltpu.sync_copy(ei_hbm.at[1, pl.ds(base, CHUNK)], didx)
        for b in range(NRING):
            pltpu.async_copy(x_hbm.at[sidx.at[b]], rows.at[b], gsems[b])

        def _pair(p, carry):
            for b in range(NRING):
                j = p * NRING + b
                pltpu.make_async_copy(x_hbm.at[sidx.at[j]], rows.at[b],
                                      gsems[b]).wait()
                pltpu.sync_copy(rows.at[b], acc.at[didx.at[j]], add=True)
                pltpu.async_copy(x_hbm.at[sidx.at[j + NRING]], rows.at[b],
                                 gsems[b])
            return carry
        lax.fori_loop(0, CHUNK // NRING - 1, _pair, 0)
        for b in range(NRING):
            j = CHUNK - NRING + b
            pltpu.make_async_copy(x_hbm.at[sidx.at[j]], rows.at[b],
                                  gsems[b]).wait()
            pltpu.sync_copy(rows.at[b], acc.at[didx.at[j]], add=True)

    plsc.subcore_barrier()

    pltpu.sync_copy(acc.at[pl.ds(r0, ROWS_PER_TILE)],
                    out_hbm.at[pl.ds(cid * N + r0, ROWS_PER_TILE)])

    @pl.when(sid == NS - 1)
    def _pub_rem():
        pltpu.sync_copy(acc.at[pl.ds(NS * ROWS_PER_TILE, REM_ROWS)],
                        out_hbm.at[pl.ds(cid * N + NS * ROWS_PER_TILE,
                                         REM_ROWS)])


def _mlp_body(with_post_bn, p_ref, w1_ref, b1_ref, g1_ref, be1_ref,
              w2_ref, b2_ref, g3_ref, be3_ref, o_ref):
    u = p_ref[0:N, :] + p_ref[N:2 * N, :]
    t = jnp.dot(u, w1_ref[...], preferred_element_type=jnp.float32)
    t = t + b1_ref[...]
    mean = jnp.mean(t, axis=0, keepdims=True)
    var = jnp.mean((t - mean) ** 2, axis=0, keepdims=True)
    t = (t - mean) / jnp.sqrt(var + 1e-5) * g1_ref[...] + be1_ref[...]
    t = jnp.maximum(t, 0.0)
    t = jnp.dot(t, w2_ref[...], preferred_element_type=jnp.float32)
    t = t + b2_ref[...]
    if with_post_bn:
        mean = jnp.mean(t, axis=0, keepdims=True)
        var = jnp.mean((t - mean) ** 2, axis=0, keepdims=True)
        t = (t - mean) / jnp.sqrt(var + 1e-5) * g3_ref[...] + be3_ref[...]
        t = jnp.maximum(t, 0.0)
    o_ref[...] = t


def _mlp_tc(parts, w1, b1, g1, be1, w2, b2, g3, be3, with_post_bn):
    r2 = lambda v: v.reshape(1, D)
    return pl.pallas_call(
        functools.partial(_mlp_body, with_post_bn),
        out_shape=jax.ShapeDtypeStruct((N, D), jnp.float32),
    )(parts, w1, r2(b1), r2(g1), r2(be1), w2, r2(b2), r2(g3), r2(be3))


def kernel(x, edge_index, w0_1, b0_1, g0_1, be0_1, w0_2, b0_2, g0_3, be0_3,
           w1_1, b1_1, g1_1, be1_1, w1_2, b1_2):
    ei = edge_index.reshape(2, NBLK, EB)
    parts0 = _segsum_sc(x, ei)
    h = _mlp_tc(parts0, w0_1, b0_1, g0_1, be0_1, w0_2, b0_2, g0_3, be0_3,
                with_post_bn=True)
    parts1 = _segsum_sc(h, ei)
    out = _mlp_tc(parts1, w1_1, b1_1, g1_1, be1_1, w1_2, b1_2, g1_1, be1_1,
                  with_post_bn=False)
    return out
